# baseline (speedup 1.0000x reference)
import functools
import math

import jax
import jax.numpy as jnp
from jax.experimental import pallas as pl
from jax.experimental.pallas import tpu as pltpu
from jax.experimental.pallas import tpu_sc as plsc

NH = 8
HD = 32
TK = 2
QC = 512
RC = 512

_F32 = jnp.float32


def _embed_body(xs_ref, win_ref, bin_ref, h_ref):
    h_ref[0] = (
        jnp.dot(xs_ref[0], win_ref[...], preferred_element_type=_F32)
        + bin_ref[...]
    )


def _qkv_body(h_ref, wq_ref, bq_ref, wk_ref, bk_ref, wv_ref, bv_ref,
              q_ref, k_ref, v_ref):
    h = h_ref[0]
    q = jnp.dot(h, wq_ref[...], preferred_element_type=_F32) + bq_ref[...]
    k = jnp.dot(h, wk_ref[...], preferred_element_type=_F32) + bk_ref[...]
    v = jnp.dot(h, wv_ref[...], preferred_element_type=_F32) + bv_ref[...]
    for i in range(NH):
        sl = slice(i * HD, (i + 1) * HD)
        q_ref[i] = q[:, sl]
        k_ref[i] = k[:, sl]
        v_ref[i] = v[:, sl]


def _attn_body(q_ref, k_ref, v_ref, o_ref):
    scale = 1.0 / math.sqrt(float(HD))
    s = jax.lax.dot_general(
        q_ref[0].astype(jnp.bfloat16), k_ref[0].astype(jnp.bfloat16),
        (((1,), (1,)), ((), ())),
        preferred_element_type=_F32,
    ) * scale
    p = jax.nn.softmax(s, axis=-1)
    o_ref[0] = jnp.dot(p.astype(jnp.bfloat16), v_ref[0].astype(jnp.bfloat16),
                       preferred_element_type=_F32)


def _layer_norm(x, g, b, eps=1e-5):
    mu = jnp.mean(x, axis=-1, keepdims=True)
    xc = x - mu
    var = jnp.mean(xc * xc, axis=-1, keepdims=True)
    return xc / jnp.sqrt(var + eps) * g + b


def _post_body(o_ref, h_ref, wo_ref, bo_ref, g1_ref, be1_ref,
               w1_ref, bf1_ref, w2_ref, bf2_ref, g2_ref, be2_ref, out_ref):
    a = jnp.concatenate([o_ref[i] for i in range(NH)], axis=-1)
    o = jnp.dot(a, wo_ref[...], preferred_element_type=_F32) + bo_ref[...]
    h1 = _layer_norm(h_ref[0] + o, g1_ref[...], be1_ref[...])
    f = jnp.dot(
        jax.nn.relu(jnp.dot(h1, w1_ref[...], preferred_element_type=_F32)
                    + bf1_ref[...]),
        w2_ref[...], preferred_element_type=_F32,
    ) + bf2_ref[...]
    out_ref[0] = _layer_norm(h1 + f, g2_ref[...], be2_ref[...])


def _head_body(h_ref, wfc_ref, bfc_ref, lg_ref):
    feats = jnp.max(h_ref[0], axis=0, keepdims=True)
    lg_ref[0] = (
        jnp.dot(feats, wfc_ref[...], preferred_element_type=_F32) + bfc_ref[...]
    )


def _router_sc(logits):
    B, E = logits.shape
    mesh = plsc.VectorSubcoreMesh(core_axis_name="c", subcore_axis_name="s")

    @functools.partial(
        pl.kernel,
        out_type=[
            jax.ShapeDtypeStruct((B, E), jnp.float32),
            jax.ShapeDtypeStruct((B, E), jnp.int32),
        ],
        mesh=mesh,
        scratch_types=[
            pltpu.VMEM((E,), jnp.float32),
            pltpu.VMEM((E,), jnp.float32),
            pltpu.VMEM((E,), jnp.int32),
        ],
        compiler_params=pltpu.CompilerParams(needs_layout_passes=False),
    )
    def krn(lg_hbm, rw_hbm, idx_hbm, lg_v, rw_v, idx_v):
        wid = jax.lax.axis_index("s") * 2 + jax.lax.axis_index("c")

        @pl.when(wid < B)
        def _():
            pltpu.sync_copy(lg_hbm.at[wid], lg_v)
            v = lg_v[...]
            io = jax.lax.iota(jnp.int32, E)
            sent = jnp.int32(E)
            l1 = jnp.max(v)
            i1 = jnp.min(jnp.where(v == l1, io, sent))
            m1 = io == i1
            v2 = jnp.where(m1, -jnp.inf, v)
            l2 = jnp.max(v2)
            i2 = jnp.min(jnp.where(v2 == l2, io, sent))
            e = jnp.exp(jnp.full((E,), l2 - l1, jnp.float32))
            denom = 1.0 + e
            w1 = 1.0 / denom
            w2 = e / denom
            zero = jnp.zeros_like(v)
            rw_v[...] = jnp.where(m1, w1, jnp.where(io == i2, w2, zero))
            idx_v[...] = jnp.where(io == 0, i1, jnp.where(io == 1, i2, 0))
            pltpu.sync_copy(rw_v, rw_hbm.at[wid])
            pltpu.sync_copy(idx_v, idx_hbm.at[wid])

    rw, idx = krn(logits)
    return rw, idx[:, :TK]


def kernel(x, W_in, b_in, Wq, bq, Wk, bk, Wv, bv, Wo, bo, g1, be1,
           W1, bf1, W2, bf2, g2, be2, Wfc, bfc):
    B, C, Ht, W = x.shape
    D = W_in.shape[1]
    L = Wq.shape[0]
    S = W
    E = Wfc.shape[1]
    dff = W1.shape[2]
    BH = B * NH

    xs = jnp.transpose(x[:, :, 0, :], (0, 2, 1))

    h = pl.pallas_call(
        _embed_body,
        grid=(B,),
        in_specs=[
            pl.BlockSpec((1, S, C), lambda b: (b, 0, 0)),
            pl.BlockSpec((C, D), lambda b: (0, 0)),
            pl.BlockSpec((1, D), lambda b: (0, 0)),
        ],
        out_specs=pl.BlockSpec((1, S, D), lambda b: (b, 0, 0)),
        out_shape=jax.ShapeDtypeStruct((B, S, D), _F32),
    )(xs, W_in, b_in.reshape(1, D))

    qkv = pl.pallas_call(
        _qkv_body,
        grid=(B, S // RC),
        in_specs=[
            pl.BlockSpec((1, RC, D), lambda b, r: (b, r, 0)),
            pl.BlockSpec((D, D), lambda b, r: (0, 0)),
            pl.BlockSpec((1, D), lambda b, r: (0, 0)),
            pl.BlockSpec((D, D), lambda b, r: (0, 0)),
            pl.BlockSpec((1, D), lambda b, r: (0, 0)),
            pl.BlockSpec((D, D), lambda b, r: (0, 0)),
            pl.BlockSpec((1, D), lambda b, r: (0, 0)),
        ],
        out_specs=[
            pl.BlockSpec((NH, RC, HD), lambda b, r: (b, r, 0)),
            pl.BlockSpec((NH, RC, HD), lambda b, r: (b, r, 0)),
            pl.BlockSpec((NH, RC, HD), lambda b, r: (b, r, 0)),
        ],
        out_shape=[jax.ShapeDtypeStruct((BH, S, HD), _F32)] * 3,
    )

    attn = pl.pallas_call(
        _attn_body,
        grid=(BH, S // QC),
        in_specs=[
            pl.BlockSpec((1, QC, HD), lambda bh, c: (bh, c, 0)),
            pl.BlockSpec((1, S, HD), lambda bh, c: (bh, 0, 0)),
            pl.BlockSpec((1, S, HD), lambda bh, c: (bh, 0, 0)),
        ],
        out_specs=pl.BlockSpec((1, QC, HD), lambda bh, c: (bh, c, 0)),
        out_shape=jax.ShapeDtypeStruct((BH, S, HD), _F32),
    )

    post = pl.pallas_call(
        _post_body,
        grid=(B, S // RC),
        in_specs=[
            pl.BlockSpec((NH, RC, HD), lambda b, r: (b, r, 0)),
            pl.BlockSpec((1, RC, D), lambda b, r: (b, r, 0)),
            pl.BlockSpec((D, D), lambda b, r: (0, 0)),
            pl.BlockSpec((1, D), lambda b, r: (0, 0)),
            pl.BlockSpec((1, D), lambda b, r: (0, 0)),
            pl.BlockSpec((1, D), lambda b, r: (0, 0)),
            pl.BlockSpec((D, dff), lambda b, r: (0, 0)),
            pl.BlockSpec((1, dff), lambda b, r: (0, 0)),
            pl.BlockSpec((dff, D), lambda b, r: (0, 0)),
            pl.BlockSpec((1, D), lambda b, r: (0, 0)),
            pl.BlockSpec((1, D), lambda b, r: (0, 0)),
            pl.BlockSpec((1, D), lambda b, r: (0, 0)),
        ],
        out_specs=pl.BlockSpec((1, RC, D), lambda b, r: (b, r, 0)),
        out_shape=jax.ShapeDtypeStruct((B, S, D), _F32),
    )

    for i in range(L):
        q, k, v = qkv(h, Wq[i], bq[i].reshape(1, D), Wk[i], bk[i].reshape(1, D),
                      Wv[i], bv[i].reshape(1, D))
        o = attn(q, k, v)
        h = post(o, h, Wo[i], bo[i].reshape(1, D),
                 g1[i].reshape(1, D), be1[i].reshape(1, D),
                 W1[i], bf1[i].reshape(1, dff),
                 W2[i], bf2[i].reshape(1, D),
                 g2[i].reshape(1, D), be2[i].reshape(1, D))

    logits3 = pl.pallas_call(
        _head_body,
        grid=(B,),
        in_specs=[
            pl.BlockSpec((1, S, D), lambda b: (b, 0, 0)),
            pl.BlockSpec((D, E), lambda b: (0, 0)),
            pl.BlockSpec((1, E), lambda b: (0, 0)),
        ],
        out_specs=pl.BlockSpec((1, 1, E), lambda b: (b, 0, 0)),
        out_shape=jax.ShapeDtypeStruct((B, 1, E), _F32),
    )(h, Wfc, bfc.reshape(1, E))
    router_logits = logits3.reshape(B, E)

    routing_weights, top_k_indices = _router_sc(router_logits)
    return (routing_weights, top_k_indices, router_logits)


# speedup vs baseline: 1.3420x; 1.3420x over previous
import functools
import math

import jax
import jax.numpy as jnp
from jax.experimental import pallas as pl
from jax.experimental.pallas import tpu as pltpu
from jax.experimental.pallas import tpu_sc as plsc

NH = 8
HD = 32
TK = 2
QC = 1024
RC = 512

_F32 = jnp.float32
_BF16 = jnp.bfloat16


def _embed_body(xs_ref, win_ref, bin_ref, h_ref):
    h_ref[0] = (
        jnp.dot(xs_ref[0], win_ref[...], preferred_element_type=_F32)
        + bin_ref[...]
    )


def _qkv_body(h_ref, wq_ref, bq_ref, wk_ref, bk_ref, wv_ref, bv_ref,
              q_ref, k_ref, v_ref):
    h = h_ref[0].astype(_BF16)
    q = jnp.dot(h, wq_ref[...], preferred_element_type=_F32) + bq_ref[...]
    k = jnp.dot(h, wk_ref[...], preferred_element_type=_F32) + bk_ref[...]
    v = jnp.dot(h, wv_ref[...], preferred_element_type=_F32) + bv_ref[...]
    ones = jnp.ones((v.shape[0], 1), _BF16)
    for i in range(NH):
        sl = slice(i * HD, (i + 1) * HD)
        q_ref[i] = q[:, sl].astype(_BF16)
        k_ref[i] = k[:, sl].astype(_BF16)
        v_ref[i] = jnp.concatenate([v[:, sl].astype(_BF16), ones], axis=-1)


def _attn_body(q_ref, k_ref, v_ref, o_ref):
    c = math.log2(math.e) / math.sqrt(float(HD))
    s = jax.lax.dot_general(
        q_ref[0], k_ref[0], (((1,), (1,)), ((), ())),
        preferred_element_type=_F32,
    )
    m = jnp.max(s, axis=-1, keepdims=True)
    e = jnp.exp2((s - m) * c).astype(_BF16)
    oa = jnp.dot(e, v_ref[0], preferred_element_type=_F32)
    r = oa[:, HD:HD + 1]
    o_ref[0] = (oa[:, :HD] * (1.0 / r)).astype(_BF16)


def _layer_norm(x, g, b, eps=1e-5):
    mu = jnp.mean(x, axis=-1, keepdims=True)
    xc = x - mu
    var = jnp.mean(xc * xc, axis=-1, keepdims=True)
    return xc / jnp.sqrt(var + eps) * g + b


def _post_body(o_ref, h_ref, wo_ref, bo_ref, g1_ref, be1_ref,
               w1_ref, bf1_ref, w2_ref, bf2_ref, g2_ref, be2_ref, out_ref):
    a = jnp.concatenate([o_ref[i] for i in range(NH)], axis=-1)
    o = jnp.dot(a, wo_ref[...], preferred_element_type=_F32) + bo_ref[...]
    h1 = _layer_norm(h_ref[0] + o, g1_ref[...], be1_ref[...])
    f1 = jax.nn.relu(
        jnp.dot(h1.astype(_BF16), w1_ref[...], preferred_element_type=_F32)
        + bf1_ref[...]
    )
    f = jnp.dot(f1.astype(_BF16), w2_ref[...], preferred_element_type=_F32
                ) + bf2_ref[...]
    out_ref[0] = _layer_norm(h1 + f, g2_ref[...], be2_ref[...])


def _head_body(h_ref, wfc_ref, bfc_ref, lg_ref):
    feats = jnp.max(h_ref[0], axis=0, keepdims=True)
    lg_ref[0] = (
        jnp.dot(feats, wfc_ref[...], preferred_element_type=_F32) + bfc_ref[...]
    )


def _router_sc(logits):
    B, E = logits.shape
    mesh = plsc.VectorSubcoreMesh(core_axis_name="c", subcore_axis_name="s")

    @functools.partial(
        pl.kernel,
        out_type=[
            jax.ShapeDtypeStruct((B, E), jnp.float32),
            jax.ShapeDtypeStruct((B, E), jnp.int32),
        ],
        mesh=mesh,
        scratch_types=[
            pltpu.VMEM((E,), jnp.float32),
            pltpu.VMEM((E,), jnp.float32),
            pltpu.VMEM((E,), jnp.int32),
        ],
        compiler_params=pltpu.CompilerParams(needs_layout_passes=False),
    )
    def krn(lg_hbm, rw_hbm, idx_hbm, lg_v, rw_v, idx_v):
        wid = jax.lax.axis_index("s") * 2 + jax.lax.axis_index("c")

        @pl.when(wid < B)
        def _():
            pltpu.sync_copy(lg_hbm.at[wid], lg_v)
            v = lg_v[...]
            io = jax.lax.iota(jnp.int32, E)
            sent = jnp.int32(E)
            l1 = jnp.max(v)
            i1 = jnp.min(jnp.where(v == l1, io, sent))
            m1 = io == i1
            v2 = jnp.where(m1, -jnp.inf, v)
            l2 = jnp.max(v2)
            i2 = jnp.min(jnp.where(v2 == l2, io, sent))
            e = jnp.exp(jnp.full((E,), l2 - l1, jnp.float32))
            denom = 1.0 + e
            w1 = 1.0 / denom
            w2 = e / denom
            zero = jnp.zeros_like(v)
            rw_v[...] = jnp.where(m1, w1, jnp.where(io == i2, w2, zero))
            idx_v[...] = jnp.where(io == 0, i1, jnp.where(io == 1, i2, 0))
            pltpu.sync_copy(rw_v, rw_hbm.at[wid])
            pltpu.sync_copy(idx_v, idx_hbm.at[wid])

    rw, idx = krn(logits)
    return rw, idx[:, :TK]


def kernel(x, W_in, b_in, Wq, bq, Wk, bk, Wv, bv, Wo, bo, g1, be1,
           W1, bf1, W2, bf2, g2, be2, Wfc, bfc):
    B, C, Ht, W = x.shape
    D = W_in.shape[1]
    L = Wq.shape[0]
    S = W
    E = Wfc.shape[1]
    dff = W1.shape[2]
    BH = B * NH

    xs = jnp.transpose(x[:, :, 0, :], (0, 2, 1))
    Wq_b, Wk_b, Wv_b = (w.astype(_BF16) for w in (Wq, Wk, Wv))
    Wo_b, W1_b, W2_b = (w.astype(_BF16) for w in (Wo, W1, W2))

    h = pl.pallas_call(
        _embed_body,
        grid=(B,),
        in_specs=[
            pl.BlockSpec((1, S, C), lambda b: (b, 0, 0)),
            pl.BlockSpec((C, D), lambda b: (0, 0)),
            pl.BlockSpec((1, D), lambda b: (0, 0)),
        ],
        out_specs=pl.BlockSpec((1, S, D), lambda b: (b, 0, 0)),
        out_shape=jax.ShapeDtypeStruct((B, S, D), _F32),
    )(xs, W_in, b_in.reshape(1, D))

    qkv = pl.pallas_call(
        _qkv_body,
        grid=(B, S // RC),
        in_specs=[
            pl.BlockSpec((1, RC, D), lambda b, r: (b, r, 0)),
            pl.BlockSpec((D, D), lambda b, r: (0, 0)),
            pl.BlockSpec((1, D), lambda b, r: (0, 0)),
            pl.BlockSpec((D, D), lambda b, r: (0, 0)),
            pl.BlockSpec((1, D), lambda b, r: (0, 0)),
            pl.BlockSpec((D, D), lambda b, r: (0, 0)),
            pl.BlockSpec((1, D), lambda b, r: (0, 0)),
        ],
        out_specs=[
            pl.BlockSpec((NH, RC, HD), lambda b, r: (b, r, 0)),
            pl.BlockSpec((NH, RC, HD), lambda b, r: (b, r, 0)),
            pl.BlockSpec((NH, RC, HD + 1), lambda b, r: (b, r, 0)),
        ],
        out_shape=[
            jax.ShapeDtypeStruct((BH, S, HD), _BF16),
            jax.ShapeDtypeStruct((BH, S, HD), _BF16),
            jax.ShapeDtypeStruct((BH, S, HD + 1), _BF16),
        ],
    )

    attn = pl.pallas_call(
        _attn_body,
        grid=(BH, S // QC),
        in_specs=[
            pl.BlockSpec((1, QC, HD), lambda bh, c: (bh, c, 0)),
            pl.BlockSpec((1, S, HD), lambda bh, c: (bh, 0, 0)),
            pl.BlockSpec((1, S, HD + 1), lambda bh, c: (bh, 0, 0)),
        ],
        out_specs=pl.BlockSpec((1, QC, HD), lambda bh, c: (bh, c, 0)),
        out_shape=jax.ShapeDtypeStruct((BH, S, HD), _BF16),
    )

    post = pl.pallas_call(
        _post_body,
        grid=(B, S // RC),
        in_specs=[
            pl.BlockSpec((NH, RC, HD), lambda b, r: (b, r, 0)),
            pl.BlockSpec((1, RC, D), lambda b, r: (b, r, 0)),
            pl.BlockSpec((D, D), lambda b, r: (0, 0)),
            pl.BlockSpec((1, D), lambda b, r: (0, 0)),
            pl.BlockSpec((1, D), lambda b, r: (0, 0)),
            pl.BlockSpec((1, D), lambda b, r: (0, 0)),
            pl.BlockSpec((D, dff), lambda b, r: (0, 0)),
            pl.BlockSpec((1, dff), lambda b, r: (0, 0)),
            pl.BlockSpec((dff, D), lambda b, r: (0, 0)),
            pl.BlockSpec((1, D), lambda b, r: (0, 0)),
            pl.BlockSpec((1, D), lambda b, r: (0, 0)),
            pl.BlockSpec((1, D), lambda b, r: (0, 0)),
        ],
        out_specs=pl.BlockSpec((1, RC, D), lambda b, r: (b, r, 0)),
        out_shape=jax.ShapeDtypeStruct((B, S, D), _F32),
    )

    for i in range(L):
        q, k, v = qkv(h, Wq_b[i], bq[i].reshape(1, D), Wk_b[i],
                      bk[i].reshape(1, D), Wv_b[i], bv[i].reshape(1, D))
        o = attn(q, k, v)
        h = post(o, h, Wo_b[i], bo[i].reshape(1, D),
                 g1[i].reshape(1, D), be1[i].reshape(1, D),
                 W1_b[i], bf1[i].reshape(1, dff),
                 W2_b[i], bf2[i].reshape(1, D),
                 g2[i].reshape(1, D), be2[i].reshape(1, D))

    logits3 = pl.pallas_call(
        _head_body,
        grid=(B,),
        in_specs=[
            pl.BlockSpec((1, S, D), lambda b: (b, 0, 0)),
            pl.BlockSpec((D, E), lambda b: (0, 0)),
            pl.BlockSpec((1, E), lambda b: (0, 0)),
        ],
        out_specs=pl.BlockSpec((1, 1, E), lambda b: (b, 0, 0)),
        out_shape=jax.ShapeDtypeStruct((B, 1, E), _F32),
    )(h, Wfc, bfc.reshape(1, E))
    router_logits = logits3.reshape(B, E)

    routing_weights, top_k_indices = _router_sc(router_logits)
    return (routing_weights, top_k_indices, router_logits)


# speedup vs baseline: 1.7317x; 1.2904x over previous
import functools
import math

import jax
import jax.numpy as jnp
from jax.experimental import pallas as pl
from jax.experimental.pallas import tpu as pltpu
from jax.experimental.pallas import tpu_sc as plsc

NH = 8
HD = 32
TK = 2
QC = 2048
SB = 256
RC = 512

_F32 = jnp.float32
_BF16 = jnp.bfloat16


def _write_qkv(h_bf, wq_ref, bq_ref, wk_ref, bk_ref, wv_ref, bv_ref,
               q_ref, k_ref, v_ref):
    q = jnp.dot(h_bf, wq_ref[...], preferred_element_type=_F32) + bq_ref[...]
    k = jnp.dot(h_bf, wk_ref[...], preferred_element_type=_F32) + bk_ref[...]
    v = jnp.dot(h_bf, wv_ref[...], preferred_element_type=_F32) + bv_ref[...]
    ones = jnp.ones((v.shape[0], 1), _BF16)
    for i in range(NH):
        sl = slice(i * HD, (i + 1) * HD)
        q_ref[i] = q[:, sl].astype(_BF16)
        k_ref[i] = k[:, sl].astype(_BF16)
        v_ref[i] = jnp.concatenate([v[:, sl].astype(_BF16), ones], axis=-1)


def _embed_qkv_body(xs_ref, win_ref, bin_ref,
                    wq_ref, bq_ref, wk_ref, bk_ref, wv_ref, bv_ref,
                    h_ref, q_ref, k_ref, v_ref):
    h = (jnp.dot(xs_ref[0], win_ref[...], preferred_element_type=_F32)
         + bin_ref[...])
    h_ref[0] = h
    _write_qkv(h.astype(_BF16), wq_ref, bq_ref, wk_ref, bk_ref, wv_ref,
               bv_ref, q_ref, k_ref, v_ref)


def _attn_body(q_ref, k_ref, v_ref, o_ref):
    c = math.log2(math.e) / math.sqrt(float(HD))
    for j in range(QC // SB):
        rows = slice(j * SB, (j + 1) * SB)
        s = jax.lax.dot_general(
            q_ref[0, rows, :], k_ref[0], (((1,), (1,)), ((), ())),
            preferred_element_type=_F32,
        )
        m = jnp.max(s, axis=-1, keepdims=True)
        e = jnp.exp2(((s - m) * c).astype(_BF16))
        oa = jnp.dot(e, v_ref[0], preferred_element_type=_F32)
        r = oa[:, HD:HD + 1]
        o_ref[0, rows, :] = (oa[:, :HD] * (1.0 / r)).astype(_BF16)


def _layer_norm(x, g, b, eps=1e-5):
    mu = jnp.mean(x, axis=-1, keepdims=True)
    xc = x - mu
    var = jnp.mean(xc * xc, axis=-1, keepdims=True)
    return xc / jnp.sqrt(var + eps) * g + b


def _post_compute(o_ref, h_ref, wo_ref, bo_ref, g1_ref, be1_ref,
                  w1_ref, bf1_ref, w2_ref, bf2_ref, g2_ref, be2_ref):
    a = jnp.concatenate([o_ref[i] for i in range(NH)], axis=-1)
    o = jnp.dot(a, wo_ref[...], preferred_element_type=_F32) + bo_ref[...]
    h1 = _layer_norm(h_ref[0] + o, g1_ref[...], be1_ref[...])
    f1 = jax.nn.relu(
        jnp.dot(h1.astype(_BF16), w1_ref[...], preferred_element_type=_F32)
        + bf1_ref[...]
    )
    f = jnp.dot(f1.astype(_BF16), w2_ref[...], preferred_element_type=_F32
                ) + bf2_ref[...]
    return _layer_norm(h1 + f, g2_ref[...], be2_ref[...])


def _post_qkv_body(o_ref, h_ref, wo_ref, bo_ref, g1_ref, be1_ref,
                   w1_ref, bf1_ref, w2_ref, bf2_ref, g2_ref, be2_ref,
                   wq_ref, bq_ref, wk_ref, bk_ref, wv_ref, bv_ref,
                   out_ref, q_ref, k_ref, v_ref):
    h2 = _post_compute(o_ref, h_ref, wo_ref, bo_ref, g1_ref, be1_ref,
                       w1_ref, bf1_ref, w2_ref, bf2_ref, g2_ref, be2_ref)
    out_ref[0] = h2
    _write_qkv(h2.astype(_BF16), wq_ref, bq_ref, wk_ref, bk_ref, wv_ref,
               bv_ref, q_ref, k_ref, v_ref)


def _post_head_body(o_ref, h_ref, wo_ref, bo_ref, g1_ref, be1_ref,
                    w1_ref, bf1_ref, w2_ref, bf2_ref, g2_ref, be2_ref,
                    wfc_ref, bfc_ref, lg_ref, acc_ref):
    h2 = _post_compute(o_ref, h_ref, wo_ref, bo_ref, g1_ref, be1_ref,
                       w1_ref, bf1_ref, w2_ref, bf2_ref, g2_ref, be2_ref)
    fm = jnp.max(h2, axis=0, keepdims=True)
    r = pl.program_id(1)
    nr = pl.num_programs(1)

    @pl.when(r == 0)
    def _():
        acc_ref[...] = fm

    @pl.when(r > 0)
    def _():
        acc_ref[...] = jnp.maximum(acc_ref[...], fm)

    @pl.when(r == nr - 1)
    def _():
        lg_ref[0] = (
            jnp.dot(acc_ref[...], wfc_ref[...], preferred_element_type=_F32)
            + bfc_ref[...]
        )


def _router_sc(logits):
    B, E = logits.shape
    mesh = plsc.VectorSubcoreMesh(core_axis_name="c", subcore_axis_name="s")

    @functools.partial(
        pl.kernel,
        out_type=[
            jax.ShapeDtypeStruct((B, E), jnp.float32),
            jax.ShapeDtypeStruct((B, E), jnp.int32),
        ],
        mesh=mesh,
        scratch_types=[
            pltpu.VMEM((E,), jnp.float32),
            pltpu.VMEM((E,), jnp.float32),
            pltpu.VMEM((E,), jnp.int32),
        ],
        compiler_params=pltpu.CompilerParams(needs_layout_passes=False),
    )
    def krn(lg_hbm, rw_hbm, idx_hbm, lg_v, rw_v, idx_v):
        wid = jax.lax.axis_index("s") * 2 + jax.lax.axis_index("c")

        @pl.when(wid < B)
        def _():
            pltpu.sync_copy(lg_hbm.at[wid], lg_v)
            v = lg_v[...]
            io = jax.lax.iota(jnp.int32, E)
            sent = jnp.int32(E)
            l1 = jnp.max(v)
            i1 = jnp.min(jnp.where(v == l1, io, sent))
            m1 = io == i1
            v2 = jnp.where(m1, -jnp.inf, v)
            l2 = jnp.max(v2)
            i2 = jnp.min(jnp.where(v2 == l2, io, sent))
            e = jnp.exp(jnp.full((E,), l2 - l1, jnp.float32))
            denom = 1.0 + e
            w1 = 1.0 / denom
            w2 = e / denom
            zero = jnp.zeros_like(v)
            rw_v[...] = jnp.where(m1, w1, jnp.where(io == i2, w2, zero))
            idx_v[...] = jnp.where(io == 0, i1, jnp.where(io == 1, i2, 0))
            pltpu.sync_copy(rw_v, rw_hbm.at[wid])
            pltpu.sync_copy(idx_v, idx_hbm.at[wid])

    rw, idx = krn(logits)
    return rw, idx[:, :TK]


def _c0(n):
    return lambda b, r: (0,) * n


def kernel(x, W_in, b_in, Wq, bq, Wk, bk, Wv, bv, Wo, bo, g1, be1,
           W1, bf1, W2, bf2, g2, be2, Wfc, bfc):
    B, C, Ht, W = x.shape
    D = W_in.shape[1]
    L = Wq.shape[0]
    S = W
    E = Wfc.shape[1]
    dff = W1.shape[2]
    BH = B * NH

    xs = jnp.transpose(x[:, :, 0, :], (0, 2, 1))
    Wq_b, Wk_b, Wv_b = (w.astype(_BF16) for w in (Wq, Wk, Wv))
    Wo_b, W1_b, W2_b = (w.astype(_BF16) for w in (Wo, W1, W2))

    row = lambda shape: pl.BlockSpec(shape, lambda b, r: (b, r, 0))
    headrow = lambda shape: pl.BlockSpec(shape, lambda b, r: (b, r, 0))
    wspec = lambda shape: pl.BlockSpec(shape, _c0(len(shape)))

    qkv_in = [
        wspec((D, D)), wspec((1, D)),
        wspec((D, D)), wspec((1, D)),
        wspec((D, D)), wspec((1, D)),
    ]
    qkv_out_specs = [
        headrow((NH, RC, HD)),
        headrow((NH, RC, HD)),
        headrow((NH, RC, HD + 1)),
    ]
    qkv_out_shape = [
        jax.ShapeDtypeStruct((BH, S, HD), _BF16),
        jax.ShapeDtypeStruct((BH, S, HD), _BF16),
        jax.ShapeDtypeStruct((BH, S, HD + 1), _BF16),
    ]
    post_in = [
        headrow((NH, RC, HD)),
        row((1, RC, D)),
        wspec((D, D)), wspec((1, D)),
        wspec((1, D)), wspec((1, D)),
        wspec((D, dff)), wspec((1, dff)),
        wspec((dff, D)), wspec((1, D)),
        wspec((1, D)), wspec((1, D)),
    ]

    embed_qkv = pl.pallas_call(
        _embed_qkv_body,
        grid=(B, S // RC),
        in_specs=[row((1, RC, C)), wspec((C, D)), wspec((1, D))] + qkv_in,
        out_specs=[row((1, RC, D))] + qkv_out_specs,
        out_shape=[jax.ShapeDtypeStruct((B, S, D), _F32)] + qkv_out_shape,
    )

    attn = pl.pallas_call(
        _attn_body,
        grid=(BH, S // QC),
        in_specs=[
            pl.BlockSpec((1, QC, HD), lambda bh, c: (bh, c, 0)),
            pl.BlockSpec((1, S, HD), lambda bh, c: (bh, 0, 0)),
            pl.BlockSpec((1, S, HD + 1), lambda bh, c: (bh, 0, 0)),
        ],
        out_specs=pl.BlockSpec((1, QC, HD), lambda bh, c: (bh, c, 0)),
        out_shape=jax.ShapeDtypeStruct((BH, S, HD), _BF16),
    )

    post_qkv = pl.pallas_call(
        _post_qkv_body,
        grid=(B, S // RC),
        in_specs=post_in + qkv_in,
        out_specs=[row((1, RC, D))] + qkv_out_specs,
        out_shape=[jax.ShapeDtypeStruct((B, S, D), _F32)] + qkv_out_shape,
    )

    post_head = pl.pallas_call(
        _post_head_body,
        grid=(B, S // RC),
        in_specs=post_in + [wspec((D, E)), wspec((1, E))],
        out_specs=pl.BlockSpec((1, 1, E), lambda b, r: (b, 0, 0)),
        out_shape=jax.ShapeDtypeStruct((B, 1, E), _F32),
        scratch_shapes=[pltpu.VMEM((1, D), _F32)],
    )

    def layer_weights(i):
        return (Wo_b[i], bo[i].reshape(1, D),
                g1[i].reshape(1, D), be1[i].reshape(1, D),
                W1_b[i], bf1[i].reshape(1, dff),
                W2_b[i], bf2[i].reshape(1, D),
                g2[i].reshape(1, D), be2[i].reshape(1, D))

    def qkv_weights(i):
        return (Wq_b[i], bq[i].reshape(1, D), Wk_b[i], bk[i].reshape(1, D),
                Wv_b[i], bv[i].reshape(1, D))

    h, q, k, v = embed_qkv(xs, W_in, b_in.reshape(1, D), *qkv_weights(0))
    for i in range(L - 1):
        o = attn(q, k, v)
        h, q, k, v = post_qkv(o, h, *layer_weights(i), *qkv_weights(i + 1))
    o = attn(q, k, v)
    logits3 = post_head(o, h, *layer_weights(L - 1), Wfc, bfc.reshape(1, E))
    router_logits = logits3.reshape(B, E)

    routing_weights, top_k_indices = _router_sc(router_logits)
    return (routing_weights, top_k_indices, router_logits)
